# Initial kernel scaffold; baseline (speedup 1.0000x reference)
import jax
import jax.numpy as jnp
from jax.experimental import pallas as pl


def kernel(x, edge_index, Wl, bl, Wr, gamma, beta, HW1, Hb1, HW2, Hb2):
    raise NotImplementedError("write your pallas kernel here")


# speedup vs baseline: 4.1211x; 4.1211x over previous
import functools

import jax
import jax.numpy as jnp
from jax import lax
from jax.experimental import pallas as pl
from jax.experimental.pallas import tpu as pltpu
from jax.experimental.pallas import tpu_sc as plsc

N, E, D, H, L, T = 10000, 320000, 128, 128, 4, 5

NC, NS = 2, 16
NW = NC * NS
CH = 128
K = 79
EPT = K * CH
E_PAD = NW * EPT
RPT = 640
R = NS * RPT
CW = 128

_mesh = plsc.VectorSubcoreMesh(core_axis_name="c", subcore_axis_name="s",
                               num_cores=NC, num_subcores=NS)


def _seg_body(h_hbm, srcs_hbm, dsts_hbm, agg_out, src_idx, dst_idx, rows,
              agg_sh, sem):
    cid = lax.axis_index("c")
    sid = lax.axis_index("s")
    wid = sid * NC + cid

    z16 = jnp.zeros((16,), jnp.float32)

    def zfill(i, _):
        for ccol in range(D // 16):
            rows[i, pl.ds(ccol * 16, 16)] = z16
        return 0

    lax.fori_loop(0, CH, zfill, 0)
    base = sid * RPT
    for r in range(RPT // CH):
        pltpu.sync_copy(rows, agg_sh.at[pl.ds(base + r * CH, CH)])
    plsc.subcore_barrier()

    pltpu.sync_copy(srcs_hbm.at[wid], src_idx)
    pltpu.sync_copy(dsts_hbm.at[wid], dst_idx)

    def chunk(j, _):
        pltpu.async_copy(h_hbm.at[src_idx.at[j]], rows, sem).wait()
        pltpu.sync_copy(rows, agg_sh.at[dst_idx.at[j]], add=True)
        return 0

    lax.fori_loop(0, K, chunk, 0)
    plsc.subcore_barrier()

    pltpu.sync_copy(agg_sh.at[pl.ds(base, RPT)], agg_out.at[cid, pl.ds(base, RPT)])


_seg = pl.kernel(
    _seg_body,
    out_type=jax.ShapeDtypeStruct((NC, R, D), jnp.float32),
    mesh=_mesh,
    scratch_types=[
        pltpu.VMEM((K, CH), jnp.int32),
        pltpu.VMEM((K, CH), jnp.int32),
        pltpu.VMEM((CH, D), jnp.float32),
        pltpu.VMEM_SHARED((R, D), jnp.float32),
        pltpu.SemaphoreType.DMA,
    ],
)


def _cnt_body(dsts_hbm, cnt_out, dst_idx, ones_v, zero_v, cnt_sh):
    cid = lax.axis_index("c")
    sid = lax.axis_index("s")
    wid = sid * NC + cid
    base = sid * RPT

    def fill(ref, val):
        def f(i, _):
            for ccol in range(CW // 16):
                ref[i, pl.ds(ccol * 16, 16)] = jnp.full((16,), val, jnp.float32)
            return 0
        lax.fori_loop(0, CH, f, 0)

    fill(zero_v, 0.0)
    fill(ones_v, 1.0)
    for r in range(RPT // CH):
        pltpu.sync_copy(zero_v, cnt_sh.at[pl.ds(base + r * CH, CH)])
    plsc.subcore_barrier()

    pltpu.sync_copy(dsts_hbm.at[wid], dst_idx)

    def chunk(j, _):
        pltpu.sync_copy(ones_v, cnt_sh.at[dst_idx.at[j]], add=True)
        return 0

    lax.fori_loop(0, K, chunk, 0)
    plsc.subcore_barrier()
    pltpu.sync_copy(cnt_sh.at[pl.ds(base, RPT)], cnt_out.at[cid, pl.ds(base, RPT)])


_cnt_kernel = pl.kernel(
    _cnt_body,
    out_type=jax.ShapeDtypeStruct((NC, R, CW), jnp.float32),
    mesh=_mesh,
    scratch_types=[
        pltpu.VMEM((K, CH), jnp.int32),
        pltpu.VMEM((CH, CW), jnp.float32),
        pltpu.VMEM((CH, CW), jnp.float32),
        pltpu.VMEM_SHARED((R, CW), jnp.float32),
    ],
)


def _tc_layer_body(agg, cnt, h, wl, wr, blr, gr, br, out):
    c = cnt[0, :N, 0:1] + cnt[1, :N, 0:1]
    inv = 1.0 / jnp.maximum(c, 1.0)
    mean = (agg[0, :N, :] + agg[1, :N, :]) * inv
    z = (jnp.dot(mean, wl[...], preferred_element_type=jnp.float32)
         + jnp.dot(h[...], wr[...], preferred_element_type=jnp.float32)
         + blr[...])
    mu = jnp.mean(z, axis=0, keepdims=True)
    var = jnp.mean((z - mu) ** 2, axis=0, keepdims=True)
    zn = (z - mu) / jnp.sqrt(var + 1e-5) * gr[...] + br[...]
    out[...] = jnp.maximum(zn, 0.0)


_tc_layer = pl.pallas_call(
    _tc_layer_body,
    out_shape=jax.ShapeDtypeStruct((N, H), jnp.float32),
)


def _tc_last_body(agg, cnt, h, wl, wr, blr, gr, br, w1c, b1c, w2bd, b2, out):
    c = cnt[0, :N, 0:1] + cnt[1, :N, 0:1]
    inv = 1.0 / jnp.maximum(c, 1.0)
    mean = (agg[0, :N, :] + agg[1, :N, :]) * inv
    z = (jnp.dot(mean, wl[...], preferred_element_type=jnp.float32)
         + jnp.dot(h[...], wr[...], preferred_element_type=jnp.float32)
         + blr[...])
    mu = jnp.mean(z, axis=0, keepdims=True)
    var = jnp.mean((z - mu) ** 2, axis=0, keepdims=True)
    zn = (z - mu) / jnp.sqrt(var + 1e-5) * gr[...] + br[...]
    hf = jnp.maximum(zn, 0.0)
    zz = jnp.maximum(
        jnp.dot(hf, w1c[...], preferred_element_type=jnp.float32) + b1c[...], 0.0)
    oo = jnp.dot(zz, w2bd[...], preferred_element_type=jnp.float32) + b2[...]
    out[...] = jax.nn.sigmoid(oo)


_tc_last = pl.pallas_call(
    _tc_last_body,
    out_shape=jax.ShapeDtypeStruct((N, 8), jnp.float32),
)


def kernel(x, edge_index, Wl, bl, Wr, gamma, beta, HW1, Hb1, HW2, Hb2):
    src = edge_index[0].astype(jnp.int32)
    dst = edge_index[1].astype(jnp.int32)
    pad = E_PAD - E
    srcs = jnp.concatenate([src, jnp.zeros((pad,), jnp.int32)]).reshape(NW, K, CH)
    dsts = jnp.concatenate([dst, jnp.full((pad,), N, jnp.int32)]).reshape(NW, K, CH)

    w1c = HW1.transpose(1, 0, 2).reshape(D, T * (H // 2))
    b1c = Hb1.reshape(1, T * (H // 2))
    w2bd = jnp.zeros((T * (H // 2), 8), jnp.float32)
    for t in range(T):
        w2bd = w2bd.at[t * (H // 2):(t + 1) * (H // 2), t].set(HW2[t, :, 0])
    b2 = jnp.concatenate([Hb2[:, 0], jnp.zeros((3,), jnp.float32)]).reshape(1, 8)

    h = x
    cnt = _cnt_kernel(dsts)
    for l in range(L):
        agg = _seg(h, srcs, dsts)
        args = (agg, cnt, h, Wl[l], Wr[l], bl[l].reshape(1, H),
                gamma[l].reshape(1, H), beta[l].reshape(1, H))
        if l < L - 1:
            h = _tc_layer(*args)
        else:
            out8 = _tc_last(*args, w1c, b1c, w2bd, b2)
    return out8[:, :T]
